# baseline (speedup 1.0000x reference)
import functools

import jax
import jax.numpy as jnp
from jax import lax
from jax.experimental import pallas as pl
from jax.experimental.pallas import tpu as pltpu

_NUM_SCRATCHES = 20
_MAX_LENGTH = 50
_COLOR = 1.0


def _scratch_points(N, H, W):
    key = jax.random.key(42)
    k1, k2, k3, k4 = jax.random.split(key, 4)
    x_start = jax.random.randint(k1, (N, _NUM_SCRATCHES), 0, W)
    y_start = jax.random.randint(k2, (N, _NUM_SCRATCHES), 0, H)
    lengths = jax.random.randint(k3, (N, _NUM_SCRATCHES), 1, _MAX_LENGTH + 1)
    lengths = lengths.astype(jnp.float32)
    angles = jax.random.uniform(k4, (N, _NUM_SCRATCHES)) * 2 * 3.14159
    x_end = x_start.astype(jnp.float32) + lengths * jnp.cos(angles)
    y_end = y_start.astype(jnp.float32) + lengths * jnp.sin(angles)
    steps = int(_MAX_LENGTH * 1.5)
    t = jnp.linspace(0.0, 1.0, steps).reshape(1, 1, steps)
    xs = x_start.astype(jnp.float32)[..., None]
    ys = y_start.astype(jnp.float32)[..., None]
    xe = x_end[..., None]
    ye = y_end[..., None]
    x_points = (xs * (1 - t) + xe * t).astype(jnp.int32)
    y_points = (ys * (1 - t) + ye * t).astype(jnp.int32)
    x_points = jnp.clip(x_points, 0, W - 1).reshape(N, -1)
    y_points = jnp.clip(y_points, 0, H - 1).reshape(N, -1)
    return x_points, y_points


@functools.cache
def _bitmask_const(N, H, W):
    with jax.ensure_compile_time_eval():
        xp, yp = _scratch_points(N, H, W)
        x1 = jnp.clip(xp + 1, 0, W - 1)
        y1 = jnp.clip(yp + 1, 0, H - 1)
        n = jnp.broadcast_to(jnp.arange(N)[:, None], xp.shape)
        flat = jnp.concatenate([
            (n * H + yp) * W + xp,
            (n * H + y1) * W + xp,
            (n * H + yp) * W + x1,
        ], axis=1).reshape(-1)
        pix = jnp.zeros((N * H * W,), jnp.uint32).at[flat].set(1)
        weights = (jnp.uint32(1) << jnp.arange(32, dtype=jnp.uint32))
        mask = jnp.sum(pix.reshape(-1, 32) * weights[None, :],
                       axis=1, dtype=jnp.uint32)
        return mask.reshape(N, H, W // 32)


def kernel(img):
    N, C, H, W = img.shape
    mask = _bitmask_const(N, H, W)

    def body(img_ref, mask_ref, out_ref):
        m = mask_ref[0]
        mm = jnp.repeat(m, 32, axis=1)
        shift = lax.broadcasted_iota(jnp.uint32, (H, W), 1) & jnp.uint32(31)
        sel = ((mm >> shift) & jnp.uint32(1)) != 0
        out_ref[0, 0] = jnp.where(sel, jnp.float32(_COLOR), img_ref[0, 0])

    return pl.pallas_call(
        body,
        grid=(N, C),
        in_specs=[
            pl.BlockSpec((1, 1, H, W), lambda n, c: (n, c, 0, 0)),
            pl.BlockSpec((1, H, W // 32), lambda n, c: (n, 0, 0)),
        ],
        out_specs=pl.BlockSpec((1, 1, H, W), lambda n, c: (n, c, 0, 0)),
        out_shape=jax.ShapeDtypeStruct((N, C, H, W), jnp.float32),
    )(img, mask)
